# Initial kernel scaffold; baseline (speedup 1.0000x reference)
import jax
import jax.numpy as jnp
from jax.experimental import pallas as pl


def kernel(inputs, embeddings):
    raise NotImplementedError("write your pallas kernel here")


# speedup vs baseline: 1.8351x; 1.8351x over previous
import functools

import jax
import jax.numpy as jnp
from jax import lax
from jax.experimental import pallas as pl
from jax.experimental.pallas import tpu as pltpu
from jax.experimental.pallas import tpu_sc as plsc

_VOCAB = 1000000
_EMBED = 64
_ROWS = 16384 * 50
_LANES = 128
_UNITS = _ROWS // _LANES
_NC = 2
_NS = 16
_NW = _NC * _NS
_UNITS_PER_W = _UNITS // _NW
_KC = 8
_CHUNKS = _UNITS_PER_W // _KC


def _sc_gather(idx2d, table):
    mesh = plsc.VectorSubcoreMesh(core_axis_name="c", subcore_axis_name="s")

    @functools.partial(
        pl.kernel,
        out_type=jax.ShapeDtypeStruct((_UNITS, _LANES, _EMBED), jnp.float32),
        mesh=mesh,
        scratch_types=[
            pltpu.VMEM((_KC, _LANES), jnp.int32),
            pltpu.VMEM((_KC, _LANES, _EMBED), jnp.float32),
            pltpu.SemaphoreType.DMA,
        ],
        compiler_params=pltpu.CompilerParams(use_tc_tiling_on_sc=False),
    )
    def k(idx_hbm, table_hbm, out_hbm, idx_v, rows_v, sem):
        wid = lax.axis_index("s") * _NC + lax.axis_index("c")
        base_u = wid * _UNITS_PER_W

        @pl.loop(0, _CHUNKS)
        def _chunk(i):
            u = base_u + i * _KC
            pltpu.sync_copy(idx_hbm.at[pl.ds(u, _KC)], idx_v)
            copies = [
                pltpu.async_copy(table_hbm.at[idx_v.at[j]], rows_v.at[j], sem)
                for j in range(_KC)
            ]
            for cp in copies:
                cp.wait()
            pltpu.sync_copy(rows_v, out_hbm.at[pl.ds(u, _KC)])

    return k(idx2d, table)


def kernel(inputs, embeddings):
    idx2d = jnp.reshape(inputs.astype(jnp.int32), (_UNITS, _LANES))
    out = _sc_gather(idx2d, embeddings)
    return jnp.reshape(out, tuple(inputs.shape) + (embeddings.shape[1],))


# speedup vs baseline: 1.8450x; 1.0054x over previous
import functools

import jax
import jax.numpy as jnp
from jax import lax
from jax.experimental import pallas as pl
from jax.experimental.pallas import tpu as pltpu
from jax.experimental.pallas import tpu_sc as plsc

_VOCAB = 1000000
_EMBED = 64
_ROWS = 16384 * 50
_LANES = 128
_UNITS = _ROWS // _LANES
_NC = 2
_NS = 16
_NW = _NC * _NS
_UNITS_PER_W = _UNITS // _NW
_KC = 4
_NBUF = 2
_CHUNKS = _UNITS_PER_W // _KC


def _sc_gather(idx2d, table):
    mesh = plsc.VectorSubcoreMesh(core_axis_name="c", subcore_axis_name="s")

    @functools.partial(
        pl.kernel,
        out_type=jax.ShapeDtypeStruct((_UNITS, _LANES, _EMBED), jnp.float32),
        mesh=mesh,
        scratch_types=[
            pltpu.VMEM((_NBUF, _KC, _LANES), jnp.int32),
            pltpu.VMEM((_NBUF, _KC, _LANES, _EMBED), jnp.float32),
            pltpu.SemaphoreType.DMA,
            pltpu.SemaphoreType.DMA,
        ],
        compiler_params=pltpu.CompilerParams(use_tc_tiling_on_sc=False),
    )
    def k(idx_hbm, table_hbm, out_hbm, idx_v, rows_v, sem0, sem1):
        sems = (sem0, sem1)
        wid = lax.axis_index("s") * _NC + lax.axis_index("c")
        base_u = wid * _UNITS_PER_W

        def load_and_fire(ci, b):
            u = base_u + ci * _KC
            pltpu.sync_copy(idx_hbm.at[pl.ds(u, _KC)], idx_v.at[b])
            for j in range(_KC):
                pltpu.async_copy(
                    table_hbm.at[idx_v.at[b].at[j]], rows_v.at[b].at[j], sems[b]
                )

        def drain_and_store(ci, b):
            for j in range(_KC):
                pltpu.make_async_copy(
                    table_hbm.at[idx_v.at[b].at[j]], rows_v.at[b].at[j], sems[b]
                ).wait()
            u = base_u + ci * _KC
            pltpu.sync_copy(rows_v.at[b], out_hbm.at[pl.ds(u, _KC)])

        for b in range(_NBUF):
            load_and_fire(b, b)

        @pl.loop(0, _CHUNKS - _NBUF, step=_NBUF)
        def _chunk(i):
            for b in range(_NBUF):
                drain_and_store(i + b, b)
                load_and_fire(i + b + _NBUF, b)

        for b in range(_NBUF):
            drain_and_store(_CHUNKS - _NBUF + b, b)

    return k(idx2d, table)


def kernel(inputs, embeddings):
    idx2d = jnp.reshape(inputs.astype(jnp.int32), (_UNITS, _LANES))
    out = _sc_gather(idx2d, embeddings)
    return jnp.reshape(out, tuple(inputs.shape) + (embeddings.shape[1],))
